# baseline (speedup 1.0000x reference)
import functools

import jax
import jax.numpy as jnp
from jax import lax
from jax.experimental import pallas as pl
from jax.experimental.pallas import tpu as pltpu
from jax.experimental.pallas import tpu_sc as plsc

N = 50000
E = 800000
H = 64
G = 128
NCH = E // 128
SUB = 16
RPS = N // SUB
ZC = 125
NPAD = 50048
QUADS = 100

_vec_mesh = plsc.VectorSubcoreMesh(core_axis_name="core", subcore_axis_name="subcore")
_sc_params = pltpu.CompilerParams(use_tc_tiling_on_sc=False)


def _msg_body(nblocks, has_edge, ep_base, ext_tabs, refs):
    if ext_tabs:
        (src_hbm, dst_hbm), refs = refs[:2], refs[2:]
        h_hbm = None
        if has_edge:
            ep_hbm, refs = refs[0], refs[1:]
        else:
            ep_hbm = None
        tabs, refs = refs[:nblocks], refs[nblocks:]
        agg_hbm, sc = refs[0], refs[1:]
    else:
        if has_edge:
            (h_hbm, src_hbm, dst_hbm, ep_hbm), rest = refs[:4], refs[4:]
        else:
            (h_hbm, src_hbm, dst_hbm), rest = refs[:3], refs[3:]
            ep_hbm = None
        agg_hbm = rest[0]
        tabs = rest[1:1 + nblocks]
        sc = rest[1 + nblocks:]
    acc, zbuf = sc[0], sc[1]
    sidx = sc[2:6]
    didx = sc[6:10]
    rows = sc[10:14]
    sem_ld = sc[14:18]
    sem_g = sc[18:22]
    sem_sc = sc[22:26]

    c = lax.axis_index("core")
    s = lax.axis_index("subcore")
    nch = jnp.where(s < NCH % SUB, NCH // SUB + 1, NCH // SUB)

    if not ext_tabs:
        for b_i in range(nblocks // 2):
            tab = (tabs[b_i * 2], tabs[b_i * 2 + 1])
            col = 32 * (c + 2 * b_i)
            for k in range(RPS // ZC):
                r0 = s * RPS + k * ZC

                @pl.when(c == 0)
                def _():
                    pltpu.async_copy(h_hbm.at[pl.ds(r0, ZC), pl.ds(col, 32)],
                                     tab[0].at[pl.ds(r0, ZC)], sem_g[k % 4])

                @pl.when(c == 1)
                def _():
                    pltpu.async_copy(h_hbm.at[pl.ds(r0, ZC), pl.ds(col, 32)],
                                     tab[1].at[pl.ds(r0, ZC)], sem_g[k % 4])
            for k in range(RPS // ZC):
                pltpu.make_async_copy(h_hbm.at[pl.ds(0, ZC), pl.ds(0, 32)],
                                      tabs[0].at[pl.ds(0, ZC)], sem_g[k % 4]).wait()

    zv = jnp.zeros((16,), jnp.float32)

    @pl.loop(0, ZC)
    def _(i):
        zbuf[i, pl.ds(0, 16)] = zv
        zbuf[i, pl.ds(16, 16)] = zv

    def load_chunk(ch, j):
        pltpu.async_copy(src_hbm.at[pl.ds(ch * 128, 128)], sidx[j], sem_ld[j])
        pltpu.async_copy(dst_hbm.at[pl.ds(ch * 128, 128)], didx[j], sem_ld[j])

    def wait_loads(j):
        pltpu.make_async_copy(src_hbm.at[pl.ds(0, 128)], sidx[j], sem_ld[j]).wait()
        pltpu.make_async_copy(dst_hbm.at[pl.ds(0, 128)], didx[j], sem_ld[j]).wait()

    for b_i in range(nblocks // 2):
        tab = (tabs[b_i * 2], tabs[b_i * 2 + 1])
        b = c + 2 * b_i
        col = 32 * b
        epcol = ep_base + col

        def load_ep(ch, j):
            if has_edge:
                pltpu.async_copy(
                    ep_hbm.at[pl.ds(ch * 128, 128), pl.ds(epcol, 32)], rows[j], sem_ld[j])

        def wait_ep(j):
            if has_edge:
                pltpu.make_async_copy(
                    ep_hbm.at[pl.ds(0, 128), pl.ds(0, 32)], rows[j], sem_ld[j]).wait()

        def gather(j, do):
            @pl.when(do & (c == 0))
            def _():
                pltpu.async_copy(tab[0].at[sidx[j]], rows[j], sem_g[j], add=has_edge)

            @pl.when(do & (c == 1))
            def _():
                pltpu.async_copy(tab[1].at[sidx[j]], rows[j], sem_g[j], add=has_edge)

        for k in range(RPS // ZC):
            pltpu.async_copy(zbuf, acc.at[pl.ds(s * RPS + k * ZC, ZC)], sem_ld[k % 4])
        for k in range(RPS // ZC):
            pltpu.make_async_copy(zbuf, acc.at[pl.ds(0, ZC)], sem_ld[k % 4]).wait()
        plsc.subcore_barrier()

        load_chunk(s, 0)
        load_ep(s, 0)

        @pl.loop(0, QUADS)
        def _(k4):
            for j in range(4):
                k = k4 * 4 + j
                ch = k * SUB + s

                @pl.when(k < nch)
                def _():
                    wait_loads(j)
                    wait_ep(j)

                gather(j, k < nch)

                @pl.when((1 <= k) & (k <= nch))
                def _():
                    jp = (j - 1) % 4
                    pltpu.make_async_copy(
                        tab[0].at[sidx[jp]], rows[jp], sem_g[jp]).wait()
                    if has_edge:
                        @pl.loop(0, 128, unroll=8)
                        def _(i):
                            for half in range(2):
                                sl = pl.ds(16 * half, 16)
                                rows[jp][i, sl] = jnp.maximum(rows[jp][i, sl], 0.0)
                    pltpu.async_copy(rows[jp], acc.at[didx[jp]], sem_sc[jp], add=True)

                jn = (j + 1) % 4

                @pl.when((3 <= k) & (k - 3 < nch))
                def _():
                    pltpu.make_async_copy(rows[jn], acc.at[didx[jn]], sem_sc[jn]).wait()

                @pl.when(k + 1 < nch)
                def _():
                    ch2 = (k + 1) * SUB + s
                    load_chunk(ch2, jn)
                    load_ep(ch2, jn)

        plsc.subcore_barrier()

        pltpu.sync_copy(
            acc.at[pl.ds(s * RPS, RPS)],
            agg_hbm.at[pl.ds(s * RPS, RPS), pl.ds(col, 32)])

        if b_i + 1 < nblocks // 2:
            plsc.subcore_barrier()


def _message_pass(h, src1, dst1, ep, nblocks, has_edge, ep_base, tabs_in=None):
    ext_tabs = tabs_in is not None

    def body(*refs):
        _msg_body(nblocks, has_edge, ep_base, ext_tabs, refs)

    out_type = [jax.ShapeDtypeStruct((N, 128), jnp.float32)]
    if not ext_tabs:
        out_type += [jax.ShapeDtypeStruct((N, 32), jnp.float32)] * nblocks
    scratch = (
        [pltpu.VMEM_SHARED((N, 32), jnp.float32)]
        + [pltpu.VMEM((ZC, 32), jnp.float32)]
        + [pltpu.VMEM((128,), jnp.int32)] * 4
        + [pltpu.VMEM((128,), jnp.int32)] * 4
        + [pltpu.VMEM((128, 32), jnp.float32)] * 4
        + [pltpu.SemaphoreType.DMA] * 12
    )
    k = pl.kernel(
        body,
        out_type=out_type,
        mesh=_vec_mesh,
        scratch_types=scratch,
        compiler_params=_sc_params,
    )
    if ext_tabs:
        args = (src1, dst1) + ((ep,) if has_edge else ()) + tuple(tabs_in)
        return k(*args)[0]
    if has_edge:
        return k(h, src1, dst1, ep)[0]
    return k(h, src1, dst1)[0]


def _eproj_body(ea_ref, w_ref, b_ref, ep_ref):
    ea = ea_ref[...]
    ep_ref[...] = jnp.dot(ea, w_ref[...], preferred_element_type=jnp.float32) + b_ref[...]


def _eproj(edge_attr, w, b):
    BE = 4000
    return pl.pallas_call(
        _eproj_body,
        grid=(E // BE,),
        in_specs=[
            pl.BlockSpec((BE, 16), lambda i: (i, 0)),
            pl.BlockSpec((16, 128), lambda i: (0, 0)),
            pl.BlockSpec((1, 128), lambda i: (0, 0)),
        ],
        out_specs=pl.BlockSpec((BE, 128), lambda i: (i, 0)),
        out_shape=jax.ShapeDtypeStruct((E, 128), jnp.float32),
    )(edge_attr, w, b)


def _mlp_body(d_in, make_tabs, h_ref, agg_ref, w1_ref, b1_ref, w2_ref, sc_ref,
              sh_ref, o_ref, *tab_refs):
    z = (h_ref[...] + agg_ref[...])[:, :d_in]
    y = jnp.maximum(jnp.dot(z, w1_ref[...], preferred_element_type=jnp.float32)
                    + b1_ref[...], 0.0)
    y = jnp.dot(y, w2_ref[...], preferred_element_type=jnp.float32)
    y = jnp.maximum(y * sc_ref[...] + sh_ref[...], 0.0)
    o_ref[...] = jnp.concatenate([y, jnp.zeros_like(y)], axis=1)
    if make_tabs:
        tab_refs[0][...] = y[:, :32]
        tab_refs[1][...] = y[:, 32:]


def _mlp(h, agg, w1, b1, w2, scale, shift, d_in, make_tabs=False):
    BN = 2000
    body = functools.partial(_mlp_body, d_in, make_tabs)
    out_specs = [pl.BlockSpec((BN, 128), lambda i: (i, 0))]
    out_shape = [jax.ShapeDtypeStruct((N, 128), jnp.float32)]
    if make_tabs:
        out_specs += [pl.BlockSpec((BN, 32), lambda i: (i, 0))] * 2
        out_shape += [jax.ShapeDtypeStruct((N, 32), jnp.float32)] * 2
    res = pl.pallas_call(
        body,
        grid=(N // BN,),
        in_specs=[
            pl.BlockSpec((BN, 128), lambda i: (i, 0)),
            pl.BlockSpec((BN, 128), lambda i: (i, 0)),
            pl.BlockSpec((d_in, H), lambda i: (0, 0)),
            pl.BlockSpec((1, H), lambda i: (0, 0)),
            pl.BlockSpec((H, H), lambda i: (0, 0)),
            pl.BlockSpec((1, H), lambda i: (0, 0)),
            pl.BlockSpec((1, H), lambda i: (0, 0)),
        ],
        out_specs=out_specs,
        out_shape=out_shape,
    )(h, agg, w1, b1, w2, scale, shift)
    return res if make_tabs else res[0]


def _cls_body(p_ref, w_ref, b_ref, o_ref):
    o_ref[...] = jnp.dot(p_ref[...], w_ref[...], preferred_element_type=jnp.float32) + b_ref[...]


def _classifier(pooled, w, b, t):
    return pl.pallas_call(
        _cls_body,
        out_shape=jax.ShapeDtypeStruct((G, t), jnp.float32),
    )(pooled, w, b)


def _sget(ref, i):
    return ref[pl.ds(i, 16)][0]


def _pool_body(h_hbm, bat_hbm, roff_hbm, out_hbm, hb, bbv, roffs, prow):
    c = lax.axis_index("core")
    s = lax.axis_index("subcore")
    w = s * 2 + c

    pltpu.sync_copy(roff_hbm, roffs)
    ninf = jnp.full((16,), -jnp.inf, jnp.float32)
    zero = jnp.zeros((16,), jnp.float32)

    for gi in range(4):
        g = w * 4 + gi
        r0 = _sget(roffs, g)
        r1 = _sget(roffs, g + 1)
        a0 = (r0 // 128) * 128
        nchunks = jnp.where(r1 > r0, lax.div(r1 - a0 + 127, 128), 0)

        def chunk(k, carry):
            base = a0 + k * 128
            pltpu.sync_copy(h_hbm.at[pl.ds(base, 128), pl.ds(0, 64)], hb)
            pltpu.sync_copy(bat_hbm.at[pl.ds(base, 128)], bbv.at[pl.ds(0, 128)])

            def row(i, accs):
                sums, maxs = accs
                mf = jnp.where(_sget(bbv, i) == g, 1.0, 0.0)
                mv = jnp.full((16,), mf, jnp.float32)
                pen = jnp.full((16,), (mf - 1.0) * 3e38, jnp.float32)
                new_sums = []
                new_maxs = []
                for j in range(4):
                    v = hb[i, pl.ds(16 * j, 16)]
                    new_sums.append(sums[j] + v * mv)
                    new_maxs.append(jnp.maximum(maxs[j], v * mv + pen))
                return (tuple(new_sums), tuple(new_maxs))

            return lax.fori_loop(0, 128, row, carry)

        init = ((zero, zero, zero, zero), (ninf, ninf, ninf, ninf))
        sums, maxs = lax.fori_loop(0, nchunks, chunk, init)
        for j in range(4):
            prow[0, pl.ds(16 * j, 16)] = sums[j]
            prow[0, pl.ds(64 + 16 * j, 16)] = maxs[j]
        pltpu.sync_copy(prow, out_hbm.at[pl.ds(g, 1)])


def _pooling(h5p, batch_pad, roff_pad):
    scratch = [
        pltpu.VMEM((128, 64), jnp.float32),
        pltpu.VMEM((144,), jnp.int32),
        pltpu.VMEM((160,), jnp.int32),
        pltpu.VMEM((1, 128), jnp.float32),
    ]
    k = pl.kernel(
        _pool_body,
        out_type=jax.ShapeDtypeStruct((G, 128), jnp.float32),
        mesh=_vec_mesh,
        scratch_types=scratch,
        compiler_params=_sc_params,
    )
    return k(h5p, batch_pad, roff_pad)


def kernel(x, edge_index, edge_attr, batch, params):
    src1 = edge_index[0]
    dst1 = edge_index[1]

    convs = params["convs"]
    bns = params["bns"]

    w0 = convs[0]["We"]
    b0 = convs[0]["be"].reshape(1, 128)
    w13 = jnp.concatenate([convs[1]["We"], convs[3]["We"]], axis=1)
    b13 = jnp.concatenate([convs[1]["be"], convs[3]["be"]]).reshape(1, 128)
    ep0 = _eproj(edge_attr, w0, b0)
    ep13 = None

    def fold(l):
        bn = bns[l]
        s = bn["gamma"] / jnp.sqrt(bn["var"] + 1e-5)
        t = (convs[l]["b2"] - bn["mean"]) * s + bn["beta"]
        return s.reshape(1, H), t.reshape(1, H)

    layer_cfg = [
        (4, True, 0, "ep0"),
        (2, True, 0, "ep13"),
        (2, False, 0, None),
        (2, True, 64, "ep13"),
        (2, False, 0, None),
    ]

    h = x
    tabs = None
    for l, (nblocks, has_edge, ep_base, ep_name) in enumerate(layer_cfg):
        d_in = 32 * nblocks
        ep = ep0 if ep_name == "ep0" else ep13
        agg = _message_pass(h, src1, dst1, ep, nblocks, has_edge, ep_base,
                            tabs_in=tabs)
        if l == 0:
            ep13 = _eproj(edge_attr, w13, b13)
        scale, shift = fold(l)
        h = _mlp(h, agg, convs[l]["W1"], convs[l]["b1"].reshape(1, H),
                 convs[l]["W2"], scale, shift, d_in)

    roff = jnp.searchsorted(batch, jnp.arange(G + 1, dtype=jnp.int32)).astype(jnp.int32)
    roff_pad = jnp.concatenate([roff, jnp.full((160 - (G + 1),), N, jnp.int32)])
    batch_pad = jnp.concatenate([batch, jnp.full((NPAD - N,), G, jnp.int32)])
    h5p = jnp.pad(h, ((0, NPAD - N), (0, 0)))
    pooled = _pooling(h5p, batch_pad, roff_pad)

    t = params["cls_W"].shape[1]
    return _classifier(pooled, params["cls_W"], params["cls_b"].reshape(1, t), t)


# speedup vs baseline: 1.6111x; 1.6111x over previous
import functools

import jax
import jax.numpy as jnp
from jax import lax
from jax.experimental import pallas as pl
from jax.experimental.pallas import tpu as pltpu
from jax.experimental.pallas import tpu_sc as plsc

N = 50000
E = 800000
H = 64
G = 128
NCH = E // 128
SUB = 16
RPS = N // SUB
ZC = 125
NPAD = 50048
QUADS = 100

_vec_mesh = plsc.VectorSubcoreMesh(core_axis_name="core", subcore_axis_name="subcore")
_sc_params = pltpu.CompilerParams(use_tc_tiling_on_sc=False)


def _msg_body(nblocks, has_edge, ep_base, ext_tabs, refs):
    if ext_tabs:
        (src_hbm, dst_hbm), refs = refs[:2], refs[2:]
        h_hbm = None
        if has_edge:
            ep_hbm, refs = refs[0], refs[1:]
        else:
            ep_hbm = None
        tabs, refs = refs[:nblocks], refs[nblocks:]
        agg_hbm, sc = refs[0], refs[1:]
    else:
        if has_edge:
            (h_hbm, src_hbm, dst_hbm, ep_hbm), rest = refs[:4], refs[4:]
        else:
            (h_hbm, src_hbm, dst_hbm), rest = refs[:3], refs[3:]
            ep_hbm = None
        agg_hbm = rest[0]
        tabs = rest[1:1 + nblocks]
        sc = rest[1 + nblocks:]
    acc, zbuf = sc[0], sc[1]
    sidx = sc[2:6]
    didx = sc[6:10]
    rows = sc[10:14]
    sem_ld = sc[14:18]
    sem_g = sc[18:22]
    sem_sc = sc[22:26]

    c = lax.axis_index("core")
    s = lax.axis_index("subcore")
    nch = jnp.where(s < NCH % SUB, NCH // SUB + 1, NCH // SUB)

    if not ext_tabs:
        for b_i in range(nblocks // 2):
            tab = (tabs[b_i * 2], tabs[b_i * 2 + 1])
            col = 32 * (c + 2 * b_i)
            for k in range(RPS // ZC):
                r0 = s * RPS + k * ZC
                pltpu.sync_copy(h_hbm.at[pl.ds(r0, ZC), pl.ds(col, 32)], zbuf)

                @pl.when(c == 0)
                def _():
                    pltpu.sync_copy(zbuf, tab[0].at[pl.ds(r0, ZC)])

                @pl.when(c == 1)
                def _():
                    pltpu.sync_copy(zbuf, tab[1].at[pl.ds(r0, ZC)])

    zv = jnp.zeros((16,), jnp.float32)

    @pl.loop(0, ZC)
    def _(i):
        zbuf[i, pl.ds(0, 16)] = zv
        zbuf[i, pl.ds(16, 16)] = zv

    def load_chunk(ch, j):
        pltpu.async_copy(src_hbm.at[pl.ds(ch * 128, 128)], sidx[j], sem_ld[j])
        pltpu.async_copy(dst_hbm.at[pl.ds(ch * 128, 128)], didx[j], sem_ld[j])

    def wait_loads(j):
        pltpu.make_async_copy(src_hbm.at[pl.ds(0, 128)], sidx[j], sem_ld[j]).wait()
        pltpu.make_async_copy(dst_hbm.at[pl.ds(0, 128)], didx[j], sem_ld[j]).wait()

    for b_i in range(nblocks // 2):
        tab = (tabs[b_i * 2], tabs[b_i * 2 + 1])
        b = c + 2 * b_i
        col = 32 * b
        epcol = ep_base + col

        def load_ep(ch, j):
            if has_edge:
                pltpu.async_copy(
                    ep_hbm.at[pl.ds(ch * 128, 128), pl.ds(epcol, 32)], rows[j], sem_ld[j])

        def wait_ep(j):
            if has_edge:
                pltpu.make_async_copy(
                    ep_hbm.at[pl.ds(0, 128), pl.ds(0, 32)], rows[j], sem_ld[j]).wait()

        def gather(j, do):
            @pl.when(do & (c == 0))
            def _():
                pltpu.async_copy(tab[0].at[sidx[j]], rows[j], sem_g[j], add=has_edge)

            @pl.when(do & (c == 1))
            def _():
                pltpu.async_copy(tab[1].at[sidx[j]], rows[j], sem_g[j], add=has_edge)

        for k in range(RPS // ZC):
            pltpu.async_copy(zbuf, acc.at[pl.ds(s * RPS + k * ZC, ZC)], sem_ld[k % 4])
        for k in range(RPS // ZC):
            pltpu.make_async_copy(zbuf, acc.at[pl.ds(0, ZC)], sem_ld[k % 4]).wait()
        plsc.subcore_barrier()

        load_chunk(s, 0)
        load_ep(s, 0)

        @pl.loop(0, QUADS)
        def _(k4):
            for j in range(4):
                k = k4 * 4 + j
                ch = k * SUB + s

                @pl.when(k < nch)
                def _():
                    wait_loads(j)
                    wait_ep(j)

                gather(j, k < nch)

                @pl.when((1 <= k) & (k <= nch))
                def _():
                    jp = (j - 1) % 4
                    pltpu.make_async_copy(
                        tab[0].at[sidx[jp]], rows[jp], sem_g[jp]).wait()
                    if has_edge:
                        @pl.loop(0, 128, unroll=8)
                        def _(i):
                            for half in range(2):
                                sl = pl.ds(16 * half, 16)
                                rows[jp][i, sl] = jnp.maximum(rows[jp][i, sl], 0.0)
                    pltpu.async_copy(rows[jp], acc.at[didx[jp]], sem_sc[jp], add=True)

                jn = (j + 1) % 4

                @pl.when((3 <= k) & (k - 3 < nch))
                def _():
                    pltpu.make_async_copy(rows[jn], acc.at[didx[jn]], sem_sc[jn]).wait()

                @pl.when(k + 1 < nch)
                def _():
                    ch2 = (k + 1) * SUB + s
                    load_chunk(ch2, jn)
                    load_ep(ch2, jn)

        plsc.subcore_barrier()

        pltpu.sync_copy(
            acc.at[pl.ds(s * RPS, RPS)],
            agg_hbm.at[pl.ds(s * RPS, RPS), pl.ds(col, 32)])

        if b_i + 1 < nblocks // 2:
            plsc.subcore_barrier()


def _message_pass(h, src1, dst1, ep, nblocks, has_edge, ep_base, tabs_in=None):
    ext_tabs = tabs_in is not None

    def body(*refs):
        _msg_body(nblocks, has_edge, ep_base, ext_tabs, refs)

    out_type = [jax.ShapeDtypeStruct((N, 128), jnp.float32)]
    if not ext_tabs:
        out_type += [jax.ShapeDtypeStruct((N, 32), jnp.float32)] * nblocks
    scratch = (
        [pltpu.VMEM_SHARED((N, 32), jnp.float32)]
        + [pltpu.VMEM((ZC, 32), jnp.float32)]
        + [pltpu.VMEM((128,), jnp.int32)] * 4
        + [pltpu.VMEM((128,), jnp.int32)] * 4
        + [pltpu.VMEM((128, 32), jnp.float32)] * 4
        + [pltpu.SemaphoreType.DMA] * 12
    )
    k = pl.kernel(
        body,
        out_type=out_type,
        mesh=_vec_mesh,
        scratch_types=scratch,
        compiler_params=_sc_params,
    )
    if ext_tabs:
        args = (src1, dst1) + ((ep,) if has_edge else ()) + tuple(tabs_in)
        return k(*args)[0]
    if has_edge:
        return k(h, src1, dst1, ep)[0]
    return k(h, src1, dst1)[0]


def _eproj_body(ea_ref, w_ref, b_ref, ep_ref):
    ea = ea_ref[...]
    ep_ref[...] = jnp.dot(ea, w_ref[...], preferred_element_type=jnp.float32) + b_ref[...]


def _eproj(edge_attr, w, b):
    BE = 4000
    return pl.pallas_call(
        _eproj_body,
        grid=(E // BE,),
        in_specs=[
            pl.BlockSpec((BE, 16), lambda i: (i, 0)),
            pl.BlockSpec((16, 128), lambda i: (0, 0)),
            pl.BlockSpec((1, 128), lambda i: (0, 0)),
        ],
        out_specs=pl.BlockSpec((BE, 128), lambda i: (i, 0)),
        out_shape=jax.ShapeDtypeStruct((E, 128), jnp.float32),
    )(edge_attr, w, b)


def _mlp_body(d_in, make_tabs, h_ref, agg_ref, w1_ref, b1_ref, w2_ref, sc_ref,
              sh_ref, o_ref, *tab_refs):
    z = (h_ref[...] + agg_ref[...])[:, :d_in]
    y = jnp.maximum(jnp.dot(z, w1_ref[...], preferred_element_type=jnp.float32)
                    + b1_ref[...], 0.0)
    y = jnp.dot(y, w2_ref[...], preferred_element_type=jnp.float32)
    y = jnp.maximum(y * sc_ref[...] + sh_ref[...], 0.0)
    o_ref[...] = jnp.concatenate([y, jnp.zeros_like(y)], axis=1)
    if make_tabs:
        tab_refs[0][...] = y[:, :32]
        tab_refs[1][...] = y[:, 32:]


def _mlp(h, agg, w1, b1, w2, scale, shift, d_in, make_tabs=False):
    BN = 2000
    body = functools.partial(_mlp_body, d_in, make_tabs)
    out_specs = [pl.BlockSpec((BN, 128), lambda i: (i, 0))]
    out_shape = [jax.ShapeDtypeStruct((N, 128), jnp.float32)]
    if make_tabs:
        out_specs += [pl.BlockSpec((BN, 32), lambda i: (i, 0))] * 2
        out_shape += [jax.ShapeDtypeStruct((N, 32), jnp.float32)] * 2
    res = pl.pallas_call(
        body,
        grid=(N // BN,),
        in_specs=[
            pl.BlockSpec((BN, 128), lambda i: (i, 0)),
            pl.BlockSpec((BN, 128), lambda i: (i, 0)),
            pl.BlockSpec((d_in, H), lambda i: (0, 0)),
            pl.BlockSpec((1, H), lambda i: (0, 0)),
            pl.BlockSpec((H, H), lambda i: (0, 0)),
            pl.BlockSpec((1, H), lambda i: (0, 0)),
            pl.BlockSpec((1, H), lambda i: (0, 0)),
        ],
        out_specs=out_specs,
        out_shape=out_shape,
    )(h, agg, w1, b1, w2, scale, shift)
    return res if make_tabs else res[0]


def _cls_body(p_ref, w_ref, b_ref, o_ref):
    o_ref[...] = jnp.dot(p_ref[...], w_ref[...], preferred_element_type=jnp.float32) + b_ref[...]


def _classifier(pooled, w, b, t):
    return pl.pallas_call(
        _cls_body,
        out_shape=jax.ShapeDtypeStruct((G, t), jnp.float32),
    )(pooled, w, b)


def _sget(ref, i):
    return ref[pl.ds(i, 16)][0]


def _pool_body(h_hbm, bat_hbm, roff_hbm, out_hbm, hb, bbv, roffs, prow):
    c = lax.axis_index("core")
    s = lax.axis_index("subcore")
    w = s * 2 + c

    pltpu.sync_copy(roff_hbm, roffs)
    ninf = jnp.full((16,), -jnp.inf, jnp.float32)
    zero = jnp.zeros((16,), jnp.float32)

    for gi in range(4):
        g = w * 4 + gi
        r0 = _sget(roffs, g)
        r1 = _sget(roffs, g + 1)
        a0 = (r0 // 128) * 128
        nchunks = jnp.where(r1 > r0, lax.div(r1 - a0 + 127, 128), 0)

        def chunk(k, carry):
            base = a0 + k * 128
            pltpu.sync_copy(h_hbm.at[pl.ds(base, 128), pl.ds(0, 64)], hb)
            pltpu.sync_copy(bat_hbm.at[pl.ds(base, 128)], bbv.at[pl.ds(0, 128)])

            def row(i, accs):
                sums, maxs = accs
                mf = jnp.where(_sget(bbv, i) == g, 1.0, 0.0)
                mv = jnp.full((16,), mf, jnp.float32)
                pen = jnp.full((16,), (mf - 1.0) * 3e38, jnp.float32)
                new_sums = []
                new_maxs = []
                for j in range(4):
                    v = hb[i, pl.ds(16 * j, 16)]
                    new_sums.append(sums[j] + v * mv)
                    new_maxs.append(jnp.maximum(maxs[j], v * mv + pen))
                return (tuple(new_sums), tuple(new_maxs))

            return lax.fori_loop(0, 128, row, carry)

        init = ((zero, zero, zero, zero), (ninf, ninf, ninf, ninf))
        sums, maxs = lax.fori_loop(0, nchunks, chunk, init)
        for j in range(4):
            prow[0, pl.ds(16 * j, 16)] = sums[j]
            prow[0, pl.ds(64 + 16 * j, 16)] = maxs[j]
        pltpu.sync_copy(prow, out_hbm.at[pl.ds(g, 1)])


def _pooling(h5p, batch_pad, roff_pad):
    scratch = [
        pltpu.VMEM((128, 64), jnp.float32),
        pltpu.VMEM((144,), jnp.int32),
        pltpu.VMEM((160,), jnp.int32),
        pltpu.VMEM((1, 128), jnp.float32),
    ]
    k = pl.kernel(
        _pool_body,
        out_type=jax.ShapeDtypeStruct((G, 128), jnp.float32),
        mesh=_vec_mesh,
        scratch_types=scratch,
        compiler_params=_sc_params,
    )
    return k(h5p, batch_pad, roff_pad)


def kernel(x, edge_index, edge_attr, batch, params):
    src1 = edge_index[0]
    dst1 = edge_index[1]

    convs = params["convs"]
    bns = params["bns"]

    w0 = convs[0]["We"]
    b0 = convs[0]["be"].reshape(1, 128)
    w13 = jnp.concatenate([convs[1]["We"], convs[3]["We"]], axis=1)
    b13 = jnp.concatenate([convs[1]["be"], convs[3]["be"]]).reshape(1, 128)
    ep0 = _eproj(edge_attr, w0, b0)
    ep13 = None

    def fold(l):
        bn = bns[l]
        s = bn["gamma"] / jnp.sqrt(bn["var"] + 1e-5)
        t = (convs[l]["b2"] - bn["mean"]) * s + bn["beta"]
        return s.reshape(1, H), t.reshape(1, H)

    layer_cfg = [
        (4, True, 0, "ep0"),
        (2, True, 0, "ep13"),
        (2, False, 0, None),
        (2, True, 64, "ep13"),
        (2, False, 0, None),
    ]

    h = x
    tabs = None
    for l, (nblocks, has_edge, ep_base, ep_name) in enumerate(layer_cfg):
        d_in = 32 * nblocks
        ep = ep0 if ep_name == "ep0" else ep13
        agg = _message_pass(h, src1, dst1, ep, nblocks, has_edge, ep_base,
                            tabs_in=tabs)
        if l == 0:
            ep13 = _eproj(edge_attr, w13, b13)
        scale, shift = fold(l)
        h = _mlp(h, agg, convs[l]["W1"], convs[l]["b1"].reshape(1, H),
                 convs[l]["W2"], scale, shift, d_in)

    roff = jnp.searchsorted(batch, jnp.arange(G + 1, dtype=jnp.int32)).astype(jnp.int32)
    roff_pad = jnp.concatenate([roff, jnp.full((160 - (G + 1),), N, jnp.int32)])
    batch_pad = jnp.concatenate([batch, jnp.full((NPAD - N,), G, jnp.int32)])
    h5p = jnp.pad(h, ((0, NPAD - N), (0, 0)))
    pooled = _pooling(h5p, batch_pad, roff_pad)

    t = params["cls_W"].shape[1]
    return _classifier(pooled, params["cls_W"], params["cls_b"].reshape(1, t), t)


# speedup vs baseline: 1.6230x; 1.0074x over previous
import functools

import jax
import jax.numpy as jnp
from jax import lax
from jax.experimental import pallas as pl
from jax.experimental.pallas import tpu as pltpu
from jax.experimental.pallas import tpu_sc as plsc

N = 50000
E = 800000
H = 64
G = 128
NCH = E // 128
SUB = 16
RPS = N // SUB
ZC = 125
NPAD = 50048
QUADS = 100

_vec_mesh = plsc.VectorSubcoreMesh(core_axis_name="core", subcore_axis_name="subcore")
_sc_params = pltpu.CompilerParams(use_tc_tiling_on_sc=False)


def _msg_body(nblocks, has_edge, ep_base, ext_tabs, refs):
    if ext_tabs:
        (src_hbm, dst_hbm), refs = refs[:2], refs[2:]
        h_hbm = None
        if has_edge:
            ep_hbm, refs = refs[0], refs[1:]
        else:
            ep_hbm = None
        tabs, refs = refs[:nblocks], refs[nblocks:]
        agg_hbm, sc = refs[0], refs[1:]
    else:
        if has_edge:
            (h_hbm, src_hbm, dst_hbm, ep_hbm), rest = refs[:4], refs[4:]
        else:
            (h_hbm, src_hbm, dst_hbm), rest = refs[:3], refs[3:]
            ep_hbm = None
        agg_hbm = rest[0]
        tabs = rest[1:1 + nblocks]
        sc = rest[1 + nblocks:]
    acc, zbuf = sc[0], sc[1]
    sidx = sc[2:6]
    didx = sc[6:10]
    rows = sc[10:14]
    sem_ld = sc[14:18]
    sem_g = sc[18:22]
    sem_sc = sc[22:26]

    c = lax.axis_index("core")
    s = lax.axis_index("subcore")
    nch = jnp.where(s < NCH % SUB, NCH // SUB + 1, NCH // SUB)

    if not ext_tabs:
        for b_i in range(nblocks // 2):
            tab = (tabs[b_i * 2], tabs[b_i * 2 + 1])
            col = 32 * (c + 2 * b_i)
            for k in range(RPS // ZC):
                r0 = s * RPS + k * ZC
                pltpu.sync_copy(h_hbm.at[pl.ds(r0, ZC), pl.ds(col, 32)], zbuf)

                @pl.when(c == 0)
                def _():
                    pltpu.sync_copy(zbuf, tab[0].at[pl.ds(r0, ZC)])

                @pl.when(c == 1)
                def _():
                    pltpu.sync_copy(zbuf, tab[1].at[pl.ds(r0, ZC)])

    zv = jnp.zeros((16,), jnp.float32)

    @pl.loop(0, ZC)
    def _(i):
        zbuf[i, pl.ds(0, 16)] = zv
        zbuf[i, pl.ds(16, 16)] = zv

    def load_chunk(ch, j):
        pltpu.async_copy(src_hbm.at[pl.ds(ch * 128, 128)], sidx[j], sem_ld[j])
        pltpu.async_copy(dst_hbm.at[pl.ds(ch * 128, 128)], didx[j], sem_ld[j])

    def wait_loads(j):
        pltpu.make_async_copy(src_hbm.at[pl.ds(0, 128)], sidx[j], sem_ld[j]).wait()
        pltpu.make_async_copy(dst_hbm.at[pl.ds(0, 128)], didx[j], sem_ld[j]).wait()

    for b_i in range(nblocks // 2):
        tab = (tabs[b_i * 2], tabs[b_i * 2 + 1])
        b = c + 2 * b_i
        col = 32 * b
        epcol = ep_base + col

        def load_ep(ch, j):
            if has_edge:
                pltpu.async_copy(
                    ep_hbm.at[pl.ds(ch * 128, 128), pl.ds(epcol, 32)], rows[j], sem_ld[j])

        def wait_ep(j):
            if has_edge:
                pltpu.make_async_copy(
                    ep_hbm.at[pl.ds(0, 128), pl.ds(0, 32)], rows[j], sem_ld[j]).wait()

        def gather(j, do):
            @pl.when(do & (c == 0))
            def _():
                pltpu.async_copy(tab[0].at[sidx[j]], rows[j], sem_g[j], add=has_edge)

            @pl.when(do & (c == 1))
            def _():
                pltpu.async_copy(tab[1].at[sidx[j]], rows[j], sem_g[j], add=has_edge)

        for k in range(RPS // ZC):
            pltpu.async_copy(zbuf, acc.at[pl.ds(s * RPS + k * ZC, ZC)], sem_ld[k % 4])
        for k in range(RPS // ZC):
            pltpu.make_async_copy(zbuf, acc.at[pl.ds(0, ZC)], sem_ld[k % 4]).wait()
        plsc.subcore_barrier()

        load_chunk(s, 0)
        load_ep(s, 0)

        @pl.loop(0, QUADS)
        def _(k4):
            for j in range(4):
                k = k4 * 4 + j
                ch = k * SUB + s

                @pl.when(k < nch)
                def _():
                    wait_loads(j)
                    wait_ep(j)

                gather(j, k < nch)

                @pl.when((1 <= k) & (k <= nch))
                def _():
                    jp = (j - 1) % 4
                    pltpu.make_async_copy(
                        tab[0].at[sidx[jp]], rows[jp], sem_g[jp]).wait()
                    if has_edge:
                        @pl.loop(0, 128, unroll=8)
                        def _(i):
                            for half in range(2):
                                sl = pl.ds(16 * half, 16)
                                rows[jp][i, sl] = jnp.maximum(rows[jp][i, sl], 0.0)
                    pltpu.async_copy(rows[jp], acc.at[didx[jp]], sem_sc[jp], add=True)

                jn = (j + 1) % 4

                @pl.when((3 <= k) & (k - 3 < nch))
                def _():
                    pltpu.make_async_copy(rows[jn], acc.at[didx[jn]], sem_sc[jn]).wait()

                @pl.when(k + 1 < nch)
                def _():
                    ch2 = (k + 1) * SUB + s
                    load_chunk(ch2, jn)
                    load_ep(ch2, jn)

        plsc.subcore_barrier()

        pltpu.sync_copy(
            acc.at[pl.ds(s * RPS, RPS)],
            agg_hbm.at[pl.ds(s * RPS, RPS), pl.ds(col, 32)])

        if b_i + 1 < nblocks // 2:
            plsc.subcore_barrier()


def _message_pass(h, src1, dst1, ep, nblocks, has_edge, ep_base, tabs_in=None):
    ext_tabs = tabs_in is not None

    def body(*refs):
        _msg_body(nblocks, has_edge, ep_base, ext_tabs, refs)

    out_type = [jax.ShapeDtypeStruct((N, 128), jnp.float32)]
    if not ext_tabs:
        out_type += [jax.ShapeDtypeStruct((N, 32), jnp.float32)] * nblocks
    scratch = (
        [pltpu.VMEM_SHARED((N, 32), jnp.float32)]
        + [pltpu.VMEM((ZC, 32), jnp.float32)]
        + [pltpu.VMEM((128,), jnp.int32)] * 4
        + [pltpu.VMEM((128,), jnp.int32)] * 4
        + [pltpu.VMEM((128, 32), jnp.float32)] * 4
        + [pltpu.SemaphoreType.DMA] * 12
    )
    k = pl.kernel(
        body,
        out_type=out_type,
        mesh=_vec_mesh,
        scratch_types=scratch,
        compiler_params=_sc_params,
    )
    if ext_tabs:
        args = (src1, dst1) + ((ep,) if has_edge else ()) + tuple(tabs_in)
        return k(*args)[0]
    if has_edge:
        return k(h, src1, dst1, ep)[0]
    return k(h, src1, dst1)[0]


def _eproj_body(ea_ref, w0_ref, b0_ref, w13_ref, b13_ref, ep0_ref, ep13_ref):
    ea = ea_ref[...]
    ep0_ref[...] = jnp.dot(ea, w0_ref[...], preferred_element_type=jnp.float32) + b0_ref[...]
    ep13_ref[...] = jnp.dot(ea, w13_ref[...], preferred_element_type=jnp.float32) + b13_ref[...]


def _eproj(edge_attr, w0, b0, w13, b13):
    BE = 4000
    return pl.pallas_call(
        _eproj_body,
        grid=(E // BE,),
        in_specs=[
            pl.BlockSpec((BE, 16), lambda i: (i, 0)),
            pl.BlockSpec((16, 128), lambda i: (0, 0)),
            pl.BlockSpec((1, 128), lambda i: (0, 0)),
            pl.BlockSpec((16, 128), lambda i: (0, 0)),
            pl.BlockSpec((1, 128), lambda i: (0, 0)),
        ],
        out_specs=[
            pl.BlockSpec((BE, 128), lambda i: (i, 0)),
            pl.BlockSpec((BE, 128), lambda i: (i, 0)),
        ],
        out_shape=[
            jax.ShapeDtypeStruct((E, 128), jnp.float32),
            jax.ShapeDtypeStruct((E, 128), jnp.float32),
        ],
    )(edge_attr, w0, b0, w13, b13)


def _mlp_body(d_in, make_tabs, h_ref, agg_ref, w1_ref, b1_ref, w2_ref, sc_ref,
              sh_ref, o_ref, *tab_refs):
    z = (h_ref[...] + agg_ref[...])[:, :d_in]
    y = jnp.maximum(jnp.dot(z, w1_ref[...], preferred_element_type=jnp.float32)
                    + b1_ref[...], 0.0)
    y = jnp.dot(y, w2_ref[...], preferred_element_type=jnp.float32)
    y = jnp.maximum(y * sc_ref[...] + sh_ref[...], 0.0)
    o_ref[...] = jnp.concatenate([y, jnp.zeros_like(y)], axis=1)
    if make_tabs:
        tab_refs[0][...] = y[:, :32]
        tab_refs[1][...] = y[:, 32:]


def _mlp(h, agg, w1, b1, w2, scale, shift, d_in, make_tabs=False):
    BN = 2000
    body = functools.partial(_mlp_body, d_in, make_tabs)
    out_specs = [pl.BlockSpec((BN, 128), lambda i: (i, 0))]
    out_shape = [jax.ShapeDtypeStruct((N, 128), jnp.float32)]
    if make_tabs:
        out_specs += [pl.BlockSpec((BN, 32), lambda i: (i, 0))] * 2
        out_shape += [jax.ShapeDtypeStruct((N, 32), jnp.float32)] * 2
    res = pl.pallas_call(
        body,
        grid=(N // BN,),
        in_specs=[
            pl.BlockSpec((BN, 128), lambda i: (i, 0)),
            pl.BlockSpec((BN, 128), lambda i: (i, 0)),
            pl.BlockSpec((d_in, H), lambda i: (0, 0)),
            pl.BlockSpec((1, H), lambda i: (0, 0)),
            pl.BlockSpec((H, H), lambda i: (0, 0)),
            pl.BlockSpec((1, H), lambda i: (0, 0)),
            pl.BlockSpec((1, H), lambda i: (0, 0)),
        ],
        out_specs=out_specs,
        out_shape=out_shape,
    )(h, agg, w1, b1, w2, scale, shift)
    return res if make_tabs else res[0]


def _cls_body(p_ref, w_ref, b_ref, o_ref):
    o_ref[...] = jnp.dot(p_ref[...], w_ref[...], preferred_element_type=jnp.float32) + b_ref[...]


def _classifier(pooled, w, b, t):
    return pl.pallas_call(
        _cls_body,
        out_shape=jax.ShapeDtypeStruct((G, t), jnp.float32),
    )(pooled, w, b)


def _sget(ref, i):
    return ref[pl.ds(i, 16)][0]


def _pool_body(h_hbm, bat_hbm, roff_hbm, out_hbm, hb, bbv, roffs, prow):
    c = lax.axis_index("core")
    s = lax.axis_index("subcore")
    w = s * 2 + c

    pltpu.sync_copy(roff_hbm, roffs)
    ninf = jnp.full((16,), -jnp.inf, jnp.float32)
    zero = jnp.zeros((16,), jnp.float32)

    for gi in range(4):
        g = w * 4 + gi
        r0 = _sget(roffs, g)
        r1 = _sget(roffs, g + 1)
        a0 = (r0 // 128) * 128
        nchunks = jnp.where(r1 > r0, lax.div(r1 - a0 + 127, 128), 0)

        def chunk(k, carry):
            base = a0 + k * 128
            pltpu.sync_copy(h_hbm.at[pl.ds(base, 128), pl.ds(0, 64)], hb)
            pltpu.sync_copy(bat_hbm.at[pl.ds(base, 128)], bbv.at[pl.ds(0, 128)])

            def row(i, accs):
                sums, maxs = accs
                mf = jnp.where(_sget(bbv, i) == g, 1.0, 0.0)
                mv = jnp.full((16,), mf, jnp.float32)
                pen = jnp.full((16,), (mf - 1.0) * 3e38, jnp.float32)
                new_sums = []
                new_maxs = []
                for j in range(4):
                    v = hb[i, pl.ds(16 * j, 16)]
                    new_sums.append(sums[j] + v * mv)
                    new_maxs.append(jnp.maximum(maxs[j], v * mv + pen))
                return (tuple(new_sums), tuple(new_maxs))

            return lax.fori_loop(0, 128, row, carry)

        init = ((zero, zero, zero, zero), (ninf, ninf, ninf, ninf))
        sums, maxs = lax.fori_loop(0, nchunks, chunk, init)
        for j in range(4):
            prow[0, pl.ds(16 * j, 16)] = sums[j]
            prow[0, pl.ds(64 + 16 * j, 16)] = maxs[j]
        pltpu.sync_copy(prow, out_hbm.at[pl.ds(g, 1)])


def _pooling(h5p, batch_pad, roff_pad):
    scratch = [
        pltpu.VMEM((128, 64), jnp.float32),
        pltpu.VMEM((144,), jnp.int32),
        pltpu.VMEM((160,), jnp.int32),
        pltpu.VMEM((1, 128), jnp.float32),
    ]
    k = pl.kernel(
        _pool_body,
        out_type=jax.ShapeDtypeStruct((G, 128), jnp.float32),
        mesh=_vec_mesh,
        scratch_types=scratch,
        compiler_params=_sc_params,
    )
    return k(h5p, batch_pad, roff_pad)


def kernel(x, edge_index, edge_attr, batch, params):
    src1 = edge_index[0]
    dst1 = edge_index[1]

    convs = params["convs"]
    bns = params["bns"]

    w0 = convs[0]["We"]
    b0 = convs[0]["be"].reshape(1, 128)
    w13 = jnp.concatenate([convs[1]["We"], convs[3]["We"]], axis=1)
    b13 = jnp.concatenate([convs[1]["be"], convs[3]["be"]]).reshape(1, 128)
    ep0, ep13 = _eproj(edge_attr, w0, b0, w13, b13)

    def fold(l):
        bn = bns[l]
        s = bn["gamma"] / jnp.sqrt(bn["var"] + 1e-5)
        t = (convs[l]["b2"] - bn["mean"]) * s + bn["beta"]
        return s.reshape(1, H), t.reshape(1, H)

    layer_cfg = [
        (4, True, 0, "ep0"),
        (2, True, 0, "ep13"),
        (2, False, 0, None),
        (2, True, 64, "ep13"),
        (2, False, 0, None),
    ]

    h = x
    tabs = None
    for l, (nblocks, has_edge, ep_base, ep_name) in enumerate(layer_cfg):
        d_in = 32 * nblocks
        ep = ep0 if ep_name == "ep0" else ep13
        agg = _message_pass(h, src1, dst1, ep, nblocks, has_edge, ep_base,
                            tabs_in=tabs)
        scale, shift = fold(l)
        h = _mlp(h, agg, convs[l]["W1"], convs[l]["b1"].reshape(1, H),
                 convs[l]["W2"], scale, shift, d_in)

    roff = jnp.searchsorted(batch, jnp.arange(G + 1, dtype=jnp.int32)).astype(jnp.int32)
    roff_pad = jnp.concatenate([roff, jnp.full((160 - (G + 1),), N, jnp.int32)])
    batch_pad = jnp.concatenate([batch, jnp.full((NPAD - N,), G, jnp.int32)])
    h5p = jnp.pad(h, ((0, NPAD - N), (0, 0)))
    pooled = _pooling(h5p, batch_pad, roff_pad)

    t = params["cls_W"].shape[1]
    return _classifier(pooled, params["cls_W"], params["cls_b"].reshape(1, t), t)
